# baseline (speedup 1.0000x reference)
import jax
import jax.numpy as jnp
from jax import lax
from jax.experimental import pallas as pl
from jax.experimental.pallas import tpu as pltpu, tpu_sc as plsc

B, S, H = 4, 8192, 1024
N = B * S
NC, NS = 2, 16
NW = NC * NS
PER_W = N // NW
CHUNK = 32
NCHUNK = PER_W // CHUNK
TOTAL = 2 * NCHUNK


def _emb_body(text_table, feat_table, text_ids, feat_ids,
              text_out, feat_out, idx_v,
              rows0, rows1, rows2, g0, g1, g2, s0, s1, s2):
    wid = lax.axis_index("s") * NC + lax.axis_index("c")
    base = wid * PER_W
    rows = (rows0, rows1, rows2)
    gsem = (g0, g1, g2)
    ssem = (s0, s1, s2)
    tables = (text_table, feat_table)
    outs = (text_out, feat_out)
    pltpu.sync_copy(text_ids.at[pl.ds(base, PER_W)], idx_v.at[pl.ds(0, PER_W)])
    pltpu.sync_copy(feat_ids.at[pl.ds(base, PER_W)],
                    idx_v.at[pl.ds(PER_W, PER_W)])

    def start_gather(t, lch, b):
        idx_s = idx_v.at[pl.ds(t * PER_W + lch * CHUNK, CHUNK)]
        pltpu.async_copy(tables[t].at[idx_s], rows[b], gsem[b])

    def wait_gather(b):
        pltpu.make_async_copy(tables[0].at[pl.ds(0, CHUNK)], rows[b],
                              gsem[b]).wait()

    def start_scatter(t, lch, b):
        pltpu.async_copy(rows[b],
                         outs[t].at[pl.ds(base + lch * CHUNK, CHUNK)],
                         ssem[b])

    def wait_scatter(b):
        pltpu.make_async_copy(rows[b], outs[0].at[pl.ds(0, CHUNK)],
                              ssem[b]).wait()

    def steady(t_cur, t_next, g, b):
        bn = (b + 2) % 3
        wait_gather(b)
        start_scatter(t_cur, g - t_cur * NCHUNK, b)
        wait_scatter(bn)
        start_gather(t_next, (g + 2) - t_next * NCHUNK, bn)

    start_gather(0, 0, 0)
    start_gather(0, 1, 1)
    wait_gather(0)
    start_scatter(0, 0, 0)
    start_gather(0, 2, 2)
    steady(0, 0, 1, 1)
    steady(0, 0, 2, 2)

    @pl.loop(3, 30, step=3)
    def _(c):
        for k in range(3):
            steady(0, 0, c + k, k)

    steady(0, 1, 30, 0)
    steady(0, 1, 31, 1)
    steady(1, 1, 32, 2)

    @pl.loop(33, 60, step=3)
    def _(c):
        for k in range(3):
            steady(1, 1, c + k, k)

    steady(1, 1, 60, 0)
    steady(1, 1, 61, 1)
    wait_gather(2)
    start_scatter(1, 62 - NCHUNK, 2)
    wait_gather(0)
    start_scatter(1, 63 - NCHUNK, 0)
    wait_scatter(1)
    wait_scatter(2)
    wait_scatter(0)


def kernel(input_ids, feature_ids, text_table, feature_table):
    t_ids = input_ids.reshape(-1).astype(jnp.int32)
    f_ids = feature_ids.reshape(-1).astype(jnp.int32)
    mesh = plsc.VectorSubcoreMesh(core_axis_name="c", subcore_axis_name="s")
    fn = pl.kernel(
        _emb_body,
        out_type=(jax.ShapeDtypeStruct((N, H), jnp.float32),
                  jax.ShapeDtypeStruct((N, H), jnp.float32)),
        mesh=mesh,
        scratch_types=[
            pltpu.VMEM((2 * PER_W,), jnp.int32),
            pltpu.VMEM((CHUNK, H), jnp.float32),
            pltpu.VMEM((CHUNK, H), jnp.float32),
            pltpu.VMEM((CHUNK, H), jnp.float32),
            pltpu.SemaphoreType.DMA,
            pltpu.SemaphoreType.DMA,
            pltpu.SemaphoreType.DMA,
            pltpu.SemaphoreType.DMA,
            pltpu.SemaphoreType.DMA,
            pltpu.SemaphoreType.DMA,
        ],
    )
    t_out, f_out = fn(text_table, feature_table, t_ids, f_ids)
    return t_out.reshape(B, S, H), f_out.reshape(B, S, H)


# speedup vs baseline: 1.0010x; 1.0010x over previous
import jax
import jax.numpy as jnp
from jax import lax
from jax.experimental import pallas as pl
from jax.experimental.pallas import tpu as pltpu, tpu_sc as plsc

B, S, H = 4, 8192, 1024
N = B * S
NC, NS = 2, 16
NW = NC * NS
PER_W = N // NW
CHUNK = 56
NFULL = PER_W // CHUNK
TAIL = PER_W - NFULL * CHUNK


def _emb_body(text_table, feat_table, text_ids, feat_ids,
              text_out, feat_out, idx_v, rows0, rows1, gsem0, gsem1):
    wid = lax.axis_index("s") * NC + lax.axis_index("c")
    base = wid * PER_W
    rows = (rows0, rows1)
    gsem = (gsem0, gsem1)
    pltpu.sync_copy(text_ids.at[pl.ds(base, PER_W)], idx_v.at[pl.ds(0, PER_W)])
    pltpu.sync_copy(feat_ids.at[pl.ds(base, PER_W)],
                    idx_v.at[pl.ds(PER_W, PER_W)])

    for t, (table, out) in enumerate(((text_table, text_out),
                                      (feat_table, feat_out))):
        def start_gather(off, n, b):
            idx_s = idx_v.at[pl.ds(t * PER_W + off, n)]
            pltpu.async_copy(table.at[idx_s], rows[b].at[pl.ds(0, n)],
                             gsem[b])

        def wait_gather(n, b):
            pltpu.make_async_copy(table.at[pl.ds(0, n)],
                                  rows[b].at[pl.ds(0, n)], gsem[b]).wait()

        def scatter(off, n, b):
            pltpu.sync_copy(rows[b].at[pl.ds(0, n)],
                            out.at[pl.ds(base + off, n)])

        start_gather(0, CHUNK, 0)
        start_gather(CHUNK, CHUNK, 1)

        @pl.loop(0, NFULL - 2, step=2)
        def _(c):
            for b in range(2):
                ch = c + b
                wait_gather(CHUNK, b)
                scatter(ch * CHUNK, CHUNK, b)
                start_gather((ch + 2) * CHUNK, CHUNK, b)

        wait_gather(CHUNK, 0)
        scatter((NFULL - 2) * CHUNK, CHUNK, 0)
        start_gather(NFULL * CHUNK, TAIL, 0)
        wait_gather(CHUNK, 1)
        scatter((NFULL - 1) * CHUNK, CHUNK, 1)
        wait_gather(TAIL, 0)
        scatter(NFULL * CHUNK, TAIL, 0)


def kernel(input_ids, feature_ids, text_table, feature_table):
    t_ids = input_ids.reshape(-1).astype(jnp.int32)
    f_ids = feature_ids.reshape(-1).astype(jnp.int32)
    mesh = plsc.VectorSubcoreMesh(core_axis_name="c", subcore_axis_name="s")
    fn = pl.kernel(
        _emb_body,
        out_type=(jax.ShapeDtypeStruct((N, H), jnp.float32),
                  jax.ShapeDtypeStruct((N, H), jnp.float32)),
        mesh=mesh,
        scratch_types=[
            pltpu.VMEM((2 * PER_W,), jnp.int32),
            pltpu.VMEM((CHUNK, H), jnp.float32),
            pltpu.VMEM((CHUNK, H), jnp.float32),
            pltpu.SemaphoreType.DMA,
            pltpu.SemaphoreType.DMA,
        ],
    )
    t_out, f_out = fn(text_table, feature_table, t_ids, f_ids)
    return t_out.reshape(B, S, H), f_out.reshape(B, S, H)


# speedup vs baseline: 1.0085x; 1.0075x over previous
import jax
import jax.numpy as jnp
from jax import lax
from jax.experimental import pallas as pl
from jax.experimental.pallas import tpu as pltpu, tpu_sc as plsc

B, S, H = 4, 8192, 1024
N = B * S
NC, NS = 2, 16
NW = NC * NS
PER_W = N // NW
CHUNK = 32
NCHUNK = PER_W // CHUNK


def _emb_body(text_table, feat_table, text_ids, feat_ids,
              text_out, feat_out, idx_v, rows0, rows1, gsem0, gsem1):
    wid = lax.axis_index("s") * NC + lax.axis_index("c")
    base = wid * PER_W
    rows = (rows0, rows1)
    gsem = (gsem0, gsem1)
    pltpu.sync_copy(text_ids.at[pl.ds(base, PER_W)], idx_v.at[pl.ds(0, PER_W)])
    pltpu.sync_copy(feat_ids.at[pl.ds(base, PER_W)],
                    idx_v.at[pl.ds(PER_W, PER_W)])

    for t, (table, out) in enumerate(((text_table, text_out),
                                      (feat_table, feat_out))):
        def start_gather(ch, b):
            idx_s = idx_v.at[pl.ds(t * PER_W + ch * CHUNK, CHUNK)]
            pltpu.async_copy(table.at[idx_s], rows[b], gsem[b])

        def wait_gather(b):
            pltpu.make_async_copy(table.at[pl.ds(0, CHUNK)], rows[b],
                                  gsem[b]).wait()

        def scatter(ch, b):
            pltpu.sync_copy(rows[b], out.at[pl.ds(base + ch * CHUNK, CHUNK)])

        start_gather(0, 0)
        start_gather(1, 1)

        @pl.loop(0, NCHUNK - 2, step=2)
        def _(c):
            for b in range(2):
                ch = c + b
                wait_gather(b)
                scatter(ch, b)
                start_gather(ch + 2, b)

        for b in range(2):
            wait_gather(b)
            scatter(NCHUNK - 2 + b, b)


def kernel(input_ids, feature_ids, text_table, feature_table):
    t_ids = input_ids.reshape(-1).astype(jnp.int32)
    f_ids = feature_ids.reshape(-1).astype(jnp.int32)
    mesh = plsc.VectorSubcoreMesh(core_axis_name="c", subcore_axis_name="s")
    fn = pl.kernel(
        _emb_body,
        out_type=(jax.ShapeDtypeStruct((N, H), jnp.float32),
                  jax.ShapeDtypeStruct((N, H), jnp.float32)),
        mesh=mesh,
        scratch_types=[
            pltpu.VMEM((2 * PER_W,), jnp.int32),
            pltpu.VMEM((CHUNK, H), jnp.float32),
            pltpu.VMEM((CHUNK, H), jnp.float32),
            pltpu.SemaphoreType.DMA,
            pltpu.SemaphoreType.DMA,
        ],
    )
    t_out, f_out = fn(text_table, feature_table, t_ids, f_ids)
    return t_out.reshape(B, S, H), f_out.reshape(B, S, H)
